# initial kernel scaffold (unmeasured)
import jax
import jax.numpy as jnp
from jax import lax
from jax.experimental import pallas as pl
from jax.experimental.pallas import tpu as pltpu

T = 4096
D = 2048
V_LOC = 8192
NZ = 4
CH = T // NZ


def _allreduce_z(partial):

    def body(p_ref, out_ref, acc_ref, comm_ref, send_sems, recv_sems, credit_sem):
        my_x = lax.axis_index("x")
        my_y = lax.axis_index("y")
        my_z = lax.axis_index("z")
        right = (my_z + 1) % NZ
        left = (my_z - 1) % NZ

        acc_ref[...] = p_ref[...]

        for s in range(2 * (NZ - 1)):
            slot = s % 2
            if s < NZ - 1:
                cs = (my_z - s) % NZ
                cr = (my_z - s - 1) % NZ
            else:
                t = s - (NZ - 1)
                cs = (my_z + 1 - t) % NZ
                cr = (my_z - t) % NZ

            if s >= 2:
                pl.semaphore_wait(credit_sem, 1)
            rdma = pltpu.make_async_remote_copy(
                src_ref=acc_ref.at[pl.ds(cs * CH, CH)],
                dst_ref=comm_ref.at[slot],
                send_sem=send_sems.at[slot],
                recv_sem=recv_sems.at[slot],
                device_id=(my_x, my_y, right),
                device_id_type=pl.DeviceIdType.MESH,
            )
            rdma.start()
            rdma.wait()
            if s < 2 * (NZ - 1) - 2:
                pl.semaphore_signal(
                    credit_sem,
                    inc=1,
                    device_id=(my_x, my_y, left),
                    device_id_type=pl.DeviceIdType.MESH,
                )
            if s < NZ - 1:
                acc_ref[pl.ds(cr * CH, CH)] = (
                    acc_ref[pl.ds(cr * CH, CH)] + comm_ref[slot]
                )
            else:
                acc_ref[pl.ds(cr * CH, CH)] = comm_ref[slot]

        out_ref[...] = acc_ref[...].astype(jnp.float32)

    return pl.pallas_call(
        body,
        out_shape=jax.ShapeDtypeStruct((T, D), jnp.float32),
        in_specs=[pl.BlockSpec(memory_space=pltpu.VMEM)],
        out_specs=pl.BlockSpec(memory_space=pltpu.VMEM),
        scratch_shapes=[
            pltpu.VMEM((T, D), jnp.bfloat16),
            pltpu.VMEM((2, CH, D), jnp.bfloat16),
            pltpu.SemaphoreType.DMA((2,)),
            pltpu.SemaphoreType.DMA((2,)),
            pltpu.SemaphoreType.REGULAR,
        ],
        compiler_params=pltpu.CompilerParams(collective_id=0),
    )(partial)


def kernel(ids, E):
    my_z = lax.axis_index("z")
    local = ids - my_z * V_LOC
    ok = (local >= 0) & (local < V_LOC)
    safe = jnp.where(ok, local, 0)
    rows = jnp.take(E, safe, axis=0)
    partial = jnp.where(ok[:, None], rows, 0.0).astype(jnp.bfloat16)
    return _allreduce_z(partial)


# baseline (device time: 512070 ns/iter reference)
import jax
import jax.numpy as jnp
from jax import lax
from jax.experimental import pallas as pl
from jax.experimental.pallas import tpu as pltpu

T = 4096
D = 2048
V_LOC = 8192
NZ = 4
CH = T // NZ


def _allreduce_z(partial):

    def body(p_ref, out_ref, comm_ref, send_sems, recv_sems, credit_sem):
        my_x = lax.axis_index("x")
        my_y = lax.axis_index("y")
        my_z = lax.axis_index("z")
        right = (my_z + 1) % NZ
        left = (my_z - 1) % NZ

        out_ref[...] = p_ref[...]

        for s in range(2 * (NZ - 1)):
            slot = s % 2
            if s < NZ - 1:
                cs = (my_z - s) % NZ
                cr = (my_z - s - 1) % NZ
            else:
                t = s - (NZ - 1)
                cs = (my_z + 1 - t) % NZ
                cr = (my_z - t) % NZ

            if s >= 2:
                pl.semaphore_wait(credit_sem, 1)
            rdma = pltpu.make_async_remote_copy(
                src_ref=out_ref.at[pl.ds(cs * CH, CH)],
                dst_ref=comm_ref.at[slot],
                send_sem=send_sems.at[slot],
                recv_sem=recv_sems.at[slot],
                device_id=(my_x, my_y, right),
                device_id_type=pl.DeviceIdType.MESH,
            )
            rdma.start()
            rdma.wait()
            if s < 2 * (NZ - 1) - 2:
                pl.semaphore_signal(
                    credit_sem,
                    inc=1,
                    device_id=(my_x, my_y, left),
                    device_id_type=pl.DeviceIdType.MESH,
                )
            if s < NZ - 1:
                out_ref[pl.ds(cr * CH, CH)] = (
                    out_ref[pl.ds(cr * CH, CH)] + comm_ref[slot]
                )
            else:
                out_ref[pl.ds(cr * CH, CH)] = comm_ref[slot]

    return pl.pallas_call(
        body,
        out_shape=jax.ShapeDtypeStruct((T, D), jnp.bfloat16),
        in_specs=[pl.BlockSpec(memory_space=pltpu.VMEM)],
        out_specs=pl.BlockSpec(memory_space=pltpu.VMEM),
        scratch_shapes=[
            pltpu.VMEM((2, CH, D), jnp.bfloat16),
            pltpu.SemaphoreType.DMA((2,)),
            pltpu.SemaphoreType.DMA((2,)),
            pltpu.SemaphoreType.REGULAR,
        ],
        compiler_params=pltpu.CompilerParams(vmem_limit_bytes=60 * 1024 * 1024),
    )(partial)


def kernel(ids, E):
    my_z = lax.axis_index("z")
    local = ids - my_z * V_LOC
    ok = (local >= 0) & (local < V_LOC)
    safe = jnp.where(ok, local, 0)
    rows = jnp.take(E, safe, axis=0)
    partial = jnp.where(ok[:, None], rows, 0.0).astype(jnp.bfloat16)
    return _allreduce_z(partial)


# device time: 232791 ns/iter; 2.1997x vs baseline; 2.1997x over previous
import jax
import jax.numpy as jnp
from jax import lax
from jax.experimental import pallas as pl
from jax.experimental.pallas import tpu as pltpu

T = 4096
D = 2048
V_LOC = 8192
NZ = 4
GROUP = T // 4
CHA = GROUP // NZ
HALF = GROUP // 2


def _hierarchical_allreduce(partial_p):

    def body(p_ref, out_ref, comm_ref, a_send, a_recv, credit_sem, b_send, b_recv):
        x = lax.axis_index("x")
        y = lax.axis_index("y")
        z = lax.axis_index("z")
        p = 2 * x + y
        px = 2 * (1 - x) + y
        py = 2 * x + (1 - y)
        right = (z + 1) % NZ
        left = (z - 1) % NZ
        base = p * GROUP

        out_ref[pl.ds(base, GROUP)] = p_ref[...]

        for s in range(2 * (NZ - 1)):
            slot = s % 2
            if s < NZ - 1:
                cs = (z - s) % NZ
                cr = (z - s - 1) % NZ
            else:
                t = s - (NZ - 1)
                cs = (z + 1 - t) % NZ
                cr = (z - t) % NZ

            if s >= 2:
                pl.semaphore_wait(credit_sem, 1)
            rdma = pltpu.make_async_remote_copy(
                src_ref=out_ref.at[pl.ds(base + cs * CHA, CHA)],
                dst_ref=comm_ref.at[slot],
                send_sem=a_send.at[slot],
                recv_sem=a_recv.at[slot],
                device_id=(x, y, right),
                device_id_type=pl.DeviceIdType.MESH,
            )
            rdma.start()
            rdma.wait()
            if s < 2 * (NZ - 1) - 2:
                pl.semaphore_signal(
                    credit_sem,
                    inc=1,
                    device_id=(x, y, left),
                    device_id_type=pl.DeviceIdType.MESH,
                )
            if s < NZ - 1:
                out_ref[pl.ds(base + cr * CHA, CHA)] = (
                    out_ref[pl.ds(base + cr * CHA, CHA)] + comm_ref[slot]
                )
            else:
                out_ref[pl.ds(base + cr * CHA, CHA)] = comm_ref[slot]

        def xfer(g, h, target, idx):
            sl = pl.ds(g * GROUP + h * HALF, HALF)
            return pltpu.make_async_remote_copy(
                src_ref=out_ref.at[sl],
                dst_ref=out_ref.at[sl],
                send_sem=b_send.at[idx],
                recv_sem=b_recv.at[idx],
                device_id=target,
                device_id_type=pl.DeviceIdType.MESH,
            )

        xn = (1 - x, y, z)
        yn = (x, 1 - y, z)

        b1x = xfer(p, 0, xn, 0)
        b1y = xfer(p, 1, yn, 1)
        b1x.start()
        b1y.start()
        b1x.wait()
        b1y.wait()

        b2 = [
            xfer(p, 0, yn, 2),
            xfer(px, 0, yn, 3),
            xfer(p, 1, xn, 4),
            xfer(py, 1, xn, 5),
        ]
        for r in b2:
            r.start()
        for r in b2:
            r.wait()

    return pl.pallas_call(
        body,
        out_shape=jax.ShapeDtypeStruct((T, D), jnp.bfloat16),
        in_specs=[pl.BlockSpec(memory_space=pltpu.VMEM)],
        out_specs=pl.BlockSpec(memory_space=pltpu.VMEM),
        scratch_shapes=[
            pltpu.VMEM((2, CHA, D), jnp.bfloat16),
            pltpu.SemaphoreType.DMA((2,)),
            pltpu.SemaphoreType.DMA((2,)),
            pltpu.SemaphoreType.REGULAR,
            pltpu.SemaphoreType.DMA((6,)),
            pltpu.SemaphoreType.DMA((6,)),
        ],
        compiler_params=pltpu.CompilerParams(vmem_limit_bytes=60 * 1024 * 1024),
    )(partial_p)


def kernel(ids, E):
    x = lax.axis_index("x")
    y = lax.axis_index("y")
    z = lax.axis_index("z")
    p = 2 * x + y
    ids_p = lax.dynamic_slice(ids, (p * GROUP,), (GROUP,))
    local = ids_p - z * V_LOC
    ok = (local >= 0) & (local < V_LOC)
    safe = jnp.where(ok, local, 0)
    rows = jnp.take(E, safe, axis=0)
    partial_p = jnp.where(ok[:, None], rows, 0.0).astype(jnp.bfloat16)
    return _hierarchical_allreduce(partial_p)


# device time: 197346 ns/iter; 2.5948x vs baseline; 1.1796x over previous
import jax
import jax.numpy as jnp
from jax import lax
from jax.experimental import pallas as pl
from jax.experimental.pallas import tpu as pltpu

T = 4096
D = 2048
V_LOC = 8192
NZ = 4
GROUP = T // 4
CHA = GROUP // NZ
CHB = CHA // 2


def _hierarchical_allreduce(partial_p):

    def body(p_ref, out_ref, comm_ref, a_send, a_recv, credit_sem, b_send, b_recv):
        x = lax.axis_index("x")
        y = lax.axis_index("y")
        z = lax.axis_index("z")
        p = 2 * x + y
        px = 2 * (1 - x) + y
        py = 2 * x + (1 - y)
        right = (z + 1) % NZ
        left = (z - 1) % NZ
        base = p * GROUP
        xn = (1 - x, y, z)
        yn = (x, 1 - y, z)

        out_ref[pl.ds(base, GROUP)] = p_ref[...]

        def ring_step(s):
            slot = s % 2
            if s < NZ - 1:
                cs = (z - s) % NZ
                cr = (z - s - 1) % NZ
            else:
                t = s - (NZ - 1)
                cs = (z + 1 - t) % NZ
                cr = (z - t) % NZ
            if s >= 2:
                pl.semaphore_wait(credit_sem, 1)
            rdma = pltpu.make_async_remote_copy(
                src_ref=out_ref.at[pl.ds(base + cs * CHA, CHA)],
                dst_ref=comm_ref.at[slot],
                send_sem=a_send.at[slot],
                recv_sem=a_recv.at[slot],
                device_id=(x, y, right),
                device_id_type=pl.DeviceIdType.MESH,
            )
            rdma.start()
            return rdma, slot, cr

        def ring_finish(s, rdma, slot, cr):
            rdma.wait()
            if s < 2 * (NZ - 1) - 2:
                pl.semaphore_signal(
                    credit_sem,
                    inc=1,
                    device_id=(x, y, left),
                    device_id_type=pl.DeviceIdType.MESH,
                )
            if s < NZ - 1:
                out_ref[pl.ds(base + cr * CHA, CHA)] = (
                    out_ref[pl.ds(base + cr * CHA, CHA)] + comm_ref[slot]
                )
            else:
                out_ref[pl.ds(base + cr * CHA, CHA)] = comm_ref[slot]

        for s in range(NZ - 1):
            ring_finish(s, *ring_step(s))

        def xfer(g, off, target, idx):
            sl = pl.ds(g * GROUP + off, CHB)
            return pltpu.make_async_remote_copy(
                src_ref=out_ref.at[sl],
                dst_ref=out_ref.at[sl],
                send_sem=b_send.at[idx],
                recv_sem=b_recv.at[idx],
                device_id=target,
                device_id_type=pl.DeviceIdType.MESH,
            )

        def b1_start(c, k):
            b1x = xfer(p, c * CHA, xn, 6 * k)
            b1y = xfer(p, c * CHA + CHB, yn, 6 * k + 1)
            b1x.start()
            b1y.start()
            return b1x, b1y

        def b2_start(c, k):
            rs = [
                xfer(p, c * CHA, yn, 6 * k + 2),
                xfer(px, c * CHA, yn, 6 * k + 3),
                xfer(p, c * CHA + CHB, xn, 6 * k + 4),
                xfer(py, c * CHA + CHB, xn, 6 * k + 5),
            ]
            for r in rs:
                r.start()
            return rs

        own = (z + 1) % NZ
        b1_pend = b1_start(own, 0)
        prev_c, prev_k = own, 0
        b2_pend = []
        for t in range(NZ - 1):
            s = (NZ - 1) + t
            started = ring_step(s)
            b1_pend[0].wait()
            b1_pend[1].wait()
            b2_pend.extend(b2_start(prev_c, prev_k))
            ring_finish(s, *started)
            cr = started[2]
            b1_pend = b1_start(cr, t + 1)
            prev_c, prev_k = cr, t + 1
        b1_pend[0].wait()
        b1_pend[1].wait()
        b2_pend.extend(b2_start(prev_c, prev_k))
        for r in b2_pend:
            r.wait()

    return pl.pallas_call(
        body,
        out_shape=jax.ShapeDtypeStruct((T, D), jnp.bfloat16),
        in_specs=[pl.BlockSpec(memory_space=pltpu.VMEM)],
        out_specs=pl.BlockSpec(memory_space=pltpu.VMEM),
        scratch_shapes=[
            pltpu.VMEM((2, CHA, D), jnp.bfloat16),
            pltpu.SemaphoreType.DMA((2,)),
            pltpu.SemaphoreType.DMA((2,)),
            pltpu.SemaphoreType.REGULAR,
            pltpu.SemaphoreType.DMA((24,)),
            pltpu.SemaphoreType.DMA((24,)),
        ],
        compiler_params=pltpu.CompilerParams(vmem_limit_bytes=60 * 1024 * 1024),
    )(partial_p)


def kernel(ids, E):
    x = lax.axis_index("x")
    y = lax.axis_index("y")
    z = lax.axis_index("z")
    p = 2 * x + y
    ids_p = lax.dynamic_slice(ids, (p * GROUP,), (GROUP,))
    local = ids_p - z * V_LOC
    ok = (local >= 0) & (local < V_LOC)
    safe = jnp.where(ok, local, 0)
    rows = jnp.take(E, safe, axis=0)
    partial_p = jnp.where(ok[:, None], rows, 0.0).astype(jnp.bfloat16)
    return _hierarchical_allreduce(partial_p)


# device time: 172777 ns/iter; 2.9638x vs baseline; 1.1422x over previous
import jax
import jax.numpy as jnp
from jax import lax
from jax.experimental import pallas as pl
from jax.experimental.pallas import tpu as pltpu

T = 4096
D = 2048
V_LOC = 8192
NZ = 4
GROUP = T // 4
CHA = GROUP // NZ
CHB = CHA // 2


def _fused_gather_allreduce(safe_ids, mask, E):

    def body(ids_ref, mask_ref, e_ref, out_ref, gbuf, gsems,
             comm_ref, a_send, a_recv, credit_sem, b_send, b_recv):
        x = lax.axis_index("x")
        y = lax.axis_index("y")
        z = lax.axis_index("z")
        p = 2 * x + y
        px = 2 * (1 - x) + y
        py = 2 * x + (1 - y)
        right = (z + 1) % NZ
        left = (z - 1) % NZ
        base = p * GROUP
        xn = (1 - x, y, z)
        yn = (x, 1 - y, z)

        def row_copy(c, i):
            row = ids_ref[c * CHA + i]
            return pltpu.make_async_copy(
                e_ref.at[pl.ds(row, 1)],
                gbuf.at[pl.ds(c * CHA + i, 1)],
                gsems.at[c],
            )

        def gather_issue(c):
            def f(i, _):
                row_copy(c, i).start()
                return 0
            lax.fori_loop(0, CHA, f, 0)

        def gather_finish(c):
            def f(i, _):
                row_copy(c, i).wait()
                return 0
            lax.fori_loop(0, CHA, f, 0)
            out_ref[pl.ds(base + c * CHA, CHA)] = (
                gbuf[pl.ds(c * CHA, CHA)] * mask_ref[pl.ds(c * CHA, CHA)]
            ).astype(jnp.bfloat16)

        def ring_step(s):
            slot = s % 2
            if s < NZ - 1:
                cs = (z - s) % NZ
                cr = (z - s - 1) % NZ
            else:
                t = s - (NZ - 1)
                cs = (z + 1 - t) % NZ
                cr = (z - t) % NZ
            if s >= 2:
                pl.semaphore_wait(credit_sem, 1)
            rdma = pltpu.make_async_remote_copy(
                src_ref=out_ref.at[pl.ds(base + cs * CHA, CHA)],
                dst_ref=comm_ref.at[slot],
                send_sem=a_send.at[slot],
                recv_sem=a_recv.at[slot],
                device_id=(x, y, right),
                device_id_type=pl.DeviceIdType.MESH,
            )
            rdma.start()
            return rdma, slot, cr

        def ring_finish(s, rdma, slot, cr):
            rdma.wait()
            if s < 2 * (NZ - 1) - 2:
                pl.semaphore_signal(
                    credit_sem,
                    inc=1,
                    device_id=(x, y, left),
                    device_id_type=pl.DeviceIdType.MESH,
                )
            if s < NZ - 1:
                out_ref[pl.ds(base + cr * CHA, CHA)] = (
                    out_ref[pl.ds(base + cr * CHA, CHA)] + comm_ref[slot]
                )
            else:
                out_ref[pl.ds(base + cr * CHA, CHA)] = comm_ref[slot]

        c_order = [z, (z - 1) % NZ, (z - 2) % NZ, (z + 1) % NZ]
        for c in c_order:
            gather_issue(c)
        gather_finish(c_order[0])

        for s in range(NZ - 1):
            started = ring_step(s)
            gather_finish(c_order[s + 1])
            ring_finish(s, *started)

        def xfer(g, off, target, idx):
            sl = pl.ds(g * GROUP + off, CHB)
            return pltpu.make_async_remote_copy(
                src_ref=out_ref.at[sl],
                dst_ref=out_ref.at[sl],
                send_sem=b_send.at[idx],
                recv_sem=b_recv.at[idx],
                device_id=target,
                device_id_type=pl.DeviceIdType.MESH,
            )

        def b1_start(c, k):
            b1x = xfer(p, c * CHA, xn, 6 * k)
            b1y = xfer(p, c * CHA + CHB, yn, 6 * k + 1)
            b1x.start()
            b1y.start()
            return b1x, b1y

        def b2_start(c, k):
            rs = [
                xfer(p, c * CHA, yn, 6 * k + 2),
                xfer(px, c * CHA, yn, 6 * k + 3),
                xfer(p, c * CHA + CHB, xn, 6 * k + 4),
                xfer(py, c * CHA + CHB, xn, 6 * k + 5),
            ]
            for r in rs:
                r.start()
            return rs

        own = (z + 1) % NZ
        b1_pend = b1_start(own, 0)
        prev_c, prev_k = own, 0
        b2_pend = []
        for t in range(NZ - 1):
            s = (NZ - 1) + t
            started = ring_step(s)
            b1_pend[0].wait()
            b1_pend[1].wait()
            b2_pend.extend(b2_start(prev_c, prev_k))
            ring_finish(s, *started)
            cr = started[2]
            b1_pend = b1_start(cr, t + 1)
            prev_c, prev_k = cr, t + 1
        b1_pend[0].wait()
        b1_pend[1].wait()
        b2_pend.extend(b2_start(prev_c, prev_k))
        for r in b2_pend:
            r.wait()

    return pl.pallas_call(
        body,
        out_shape=jax.ShapeDtypeStruct((T, D), jnp.bfloat16),
        in_specs=[
            pl.BlockSpec(memory_space=pltpu.SMEM),
            pl.BlockSpec(memory_space=pltpu.VMEM),
            pl.BlockSpec(memory_space=pl.ANY),
        ],
        out_specs=pl.BlockSpec(memory_space=pltpu.VMEM),
        scratch_shapes=[
            pltpu.VMEM((GROUP, D), jnp.float32),
            pltpu.SemaphoreType.DMA((NZ,)),
            pltpu.VMEM((2, CHA, D), jnp.bfloat16),
            pltpu.SemaphoreType.DMA((2,)),
            pltpu.SemaphoreType.DMA((2,)),
            pltpu.SemaphoreType.REGULAR,
            pltpu.SemaphoreType.DMA((24,)),
            pltpu.SemaphoreType.DMA((24,)),
        ],
        compiler_params=pltpu.CompilerParams(vmem_limit_bytes=60 * 1024 * 1024),
    )(safe_ids, mask, E)


def kernel(ids, E):
    x = lax.axis_index("x")
    y = lax.axis_index("y")
    z = lax.axis_index("z")
    p = 2 * x + y
    ids_p = lax.dynamic_slice(ids, (p * GROUP,), (GROUP,))
    local = ids_p - z * V_LOC
    ok = (local >= 0) & (local < V_LOC)
    safe = jnp.where(ok, local, 0).astype(jnp.int32)
    mask = ok.astype(jnp.float32)[:, None]
    return _fused_gather_allreduce(safe, mask, E)
